# Initial kernel scaffold; baseline (speedup 1.0000x reference)
import jax
import jax.numpy as jnp
from jax.experimental import pallas as pl


def kernel(ids, W):
    raise NotImplementedError("write your pallas kernel here")


# speedup vs baseline: 2.9704x; 2.9704x over previous
import functools

import jax
import jax.numpy as jnp
from jax import lax
from jax.experimental import pallas as pl
from jax.experimental.pallas import tpu as pltpu
from jax.experimental.pallas import tpu_sc as plsc

B_TOTAL = 4096 * 50
EMBED = 128
NW = 32
CHUNK = 128
PER_W = B_TOTAL // NW
N_CHUNKS = PER_W // CHUNK


def _embed_kernel(ids_hbm, table_hbm, out_hbm, idx_v, rows_v, sem):
    wid = lax.axis_index("s") * 2 + lax.axis_index("c")
    base = wid * PER_W
    pltpu.sync_copy(ids_hbm.at[pl.ds(base, PER_W)], idx_v)

    def body(j, carry):
        pltpu.async_copy(
            table_hbm.at[idx_v.at[pl.ds(j * CHUNK, CHUNK)]], rows_v, sem
        ).wait()
        pltpu.sync_copy(rows_v, out_hbm.at[pl.ds(base + j * CHUNK, CHUNK)])
        return carry

    lax.fori_loop(0, N_CHUNKS, body, 0)


@jax.jit
def _embed(ids_flat, table):
    mesh = plsc.VectorSubcoreMesh(core_axis_name="c", subcore_axis_name="s")
    return pl.kernel(
        _embed_kernel,
        out_type=jax.ShapeDtypeStruct((B_TOTAL, EMBED), jnp.float32),
        mesh=mesh,
        scratch_types=[
            pltpu.VMEM((PER_W,), jnp.int32),
            pltpu.VMEM((CHUNK, EMBED), jnp.float32),
            pltpu.SemaphoreType.DMA,
        ],
    )(ids_flat, table)


def kernel(ids, W):
    orig_shape = ids.shape
    ids_flat = ids.reshape(B_TOTAL).astype(jnp.int32)
    out = _embed(ids_flat, W)
    return out.reshape(*orig_shape, EMBED)


# speedup vs baseline: 3.3163x; 1.1165x over previous
import functools

import jax
import jax.numpy as jnp
from jax import lax
from jax.experimental import pallas as pl
from jax.experimental.pallas import tpu as pltpu
from jax.experimental.pallas import tpu_sc as plsc

B_TOTAL = 4096 * 50
EMBED = 128
NW = 32
CHUNK = 128
PER_W = B_TOTAL // NW
N_CHUNKS = PER_W // CHUNK


NBUF = 5
N_GROUPS = N_CHUNKS // NBUF


def _embed_kernel(ids_hbm, table_hbm, out_hbm, idx_v, rows_v, *sems):
    gsems, osems = sems[:NBUF], sems[NBUF:]
    wid = lax.axis_index("s") * 2 + lax.axis_index("c")
    base = wid * PER_W
    pltpu.sync_copy(ids_hbm.at[pl.ds(base, PER_W)], idx_v)

    def gather(j, b):
        return pltpu.make_async_copy(
            table_hbm.at[idx_v.at[pl.ds(j * CHUNK, CHUNK)]],
            rows_v.at[b],
            gsems[b],
        )

    def writeback(j, b):
        return pltpu.make_async_copy(
            rows_v.at[b],
            out_hbm.at[pl.ds(base + j * CHUNK, CHUNK)],
            osems[b],
        )

    for b in range(NBUF):
        gather(b, b).start()

    def body(g, carry):
        j0 = g * NBUF
        for b in range(NBUF):
            gather(j0 + b, b).wait()
            writeback(j0 + b, b).start()
        for b in range(NBUF):
            writeback(j0 + b, b).wait()
            gather(j0 + NBUF + b, b).start()
        return carry

    lax.fori_loop(0, N_GROUPS - 1, body, 0)

    j0 = (N_GROUPS - 1) * NBUF
    for b in range(NBUF):
        gather(j0 + b, b).wait()
        writeback(j0 + b, b).start()
    for b in range(NBUF):
        writeback(j0 + b, b).wait()


@jax.jit
def _embed(ids_flat, table):
    mesh = plsc.VectorSubcoreMesh(core_axis_name="c", subcore_axis_name="s")
    return pl.kernel(
        _embed_kernel,
        out_type=jax.ShapeDtypeStruct((B_TOTAL, EMBED), jnp.float32),
        mesh=mesh,
        scratch_types=[
            pltpu.VMEM((PER_W,), jnp.int32),
            pltpu.VMEM((NBUF, CHUNK, EMBED), jnp.float32),
        ]
        + [pltpu.SemaphoreType.DMA] * (2 * NBUF),
    )(ids_flat, table)


def kernel(ids, W):
    orig_shape = ids.shape
    ids_flat = ids.reshape(B_TOTAL).astype(jnp.int32)
    out = _embed(ids_flat, W)
    return out.reshape(*orig_shape, EMBED)


# speedup vs baseline: 10.1486x; 3.0602x over previous
import jax
import jax.numpy as jnp
from jax import lax
from jax.experimental import pallas as pl
from jax.experimental.pallas import tpu as pltpu
from jax.experimental.pallas import tpu_sc as plsc

B_TOTAL = 4096 * 50
EMBED = 128
NW = 32
CHUNK = 128
PER_W = B_TOTAL // NW
N_CHUNKS = PER_W // CHUNK
NBUF = 5
N_GROUPS = N_CHUNKS // NBUF


def _embed_kernel(ids_hbm, table_hbm, out_hbm, idx_v, rows_v, *sems):
    gsems, osems = sems[:NBUF], sems[NBUF:]
    wid = lax.axis_index("s") * 2 + lax.axis_index("c")
    base = wid * PER_W
    pltpu.sync_copy(ids_hbm.at[pl.ds(base, PER_W)], idx_v)

    def gather(j, b):
        return pltpu.make_async_copy(
            table_hbm.at[idx_v.at[pl.ds(j * CHUNK, CHUNK)]],
            rows_v.at[b],
            gsems[b],
        )

    def writeback(j, b):
        return pltpu.make_async_copy(
            rows_v.at[b],
            out_hbm.at[pl.ds(base + j * CHUNK, CHUNK)],
            osems[b],
        )

    for b in range(NBUF):
        gather(b, b).start()

    def body(g, carry):
        j0 = g * NBUF
        for b in range(NBUF):
            gather(j0 + b, b).wait()
            writeback(j0 + b, b).start()
        for b in range(NBUF):
            writeback(j0 + b, b).wait()
            gather(j0 + NBUF + b, b).start()
        return carry

    lax.fori_loop(0, N_GROUPS - 1, body, 0)

    j0 = (N_GROUPS - 1) * NBUF
    for b in range(NBUF):
        gather(j0 + b, b).wait()
        writeback(j0 + b, b).start()
    for b in range(NBUF):
        writeback(j0 + b, b).wait()


@jax.jit
def _embed(ids_flat, table):
    mesh = plsc.VectorSubcoreMesh(core_axis_name="c", subcore_axis_name="s")
    return pl.kernel(
        _embed_kernel,
        out_type=jax.ShapeDtypeStruct((B_TOTAL, EMBED), jnp.float32),
        mesh=mesh,
        scratch_types=[
            pltpu.VMEM((PER_W,), jnp.int32),
            pltpu.VMEM((NBUF, CHUNK, EMBED), jnp.float32),
        ]
        + [pltpu.SemaphoreType.DMA] * (2 * NBUF),
    )(ids_flat, table)


def kernel(ids, W):
    batch, seq = ids.shape
    ids_t = ids.T.reshape(B_TOTAL).astype(jnp.int32)
    out = _embed(ids_t, W)
    return out.reshape(seq, batch, EMBED).transpose(1, 0, 2)


# speedup vs baseline: 10.2190x; 1.0069x over previous
import jax
import jax.numpy as jnp
from jax import lax
from jax.experimental import pallas as pl
from jax.experimental.pallas import tpu as pltpu
from jax.experimental.pallas import tpu_sc as plsc

B_TOTAL = 4096 * 50
EMBED = 128
NW = 32
CHUNK = 128
PER_W = B_TOTAL // NW
N_CHUNKS = PER_W // CHUNK
NBUF = 7


def _embed_kernel(ids_hbm, table_hbm, out_hbm, idx_v, rows_v, *sems):
    gsems, osems = sems[:NBUF], sems[NBUF:]
    wid = lax.axis_index("s") * 2 + lax.axis_index("c")
    base = wid * PER_W
    pltpu.sync_copy(ids_hbm.at[pl.ds(base, PER_W)], idx_v)

    def gather(j, b):
        return pltpu.make_async_copy(
            table_hbm.at[idx_v.at[pl.ds(j * CHUNK, CHUNK)]],
            rows_v.at[b],
            gsems[b],
        )

    def writeback(j, b):
        return pltpu.make_async_copy(
            rows_v.at[b],
            out_hbm.at[pl.ds(base + j * CHUNK, CHUNK)],
            osems[b],
        )

    for j in range(NBUF):
        gather(j, j).start()
    for j in range(N_CHUNKS):
        b = j % NBUF
        gather(j, b).wait()
        writeback(j, b).start()
        if j + NBUF < N_CHUNKS:
            writeback(j, b).wait()
            gather(j + NBUF, b).start()
    for j in range(N_CHUNKS - NBUF, N_CHUNKS):
        writeback(j, j % NBUF).wait()


@jax.jit
def _embed(ids_flat, table):
    mesh = plsc.VectorSubcoreMesh(core_axis_name="c", subcore_axis_name="s")
    return pl.kernel(
        _embed_kernel,
        out_type=jax.ShapeDtypeStruct((B_TOTAL, EMBED), jnp.float32),
        mesh=mesh,
        scratch_types=[
            pltpu.VMEM((PER_W,), jnp.int32),
            pltpu.VMEM((NBUF, CHUNK, EMBED), jnp.float32),
        ]
        + [pltpu.SemaphoreType.DMA] * (2 * NBUF),
    )(ids_flat, table)


def kernel(ids, W):
    batch, seq = ids.shape
    ids_t = ids.T.reshape(B_TOTAL).astype(jnp.int32)
    out = _embed(ids_t, W)
    return out.reshape(seq, batch, EMBED).transpose(1, 0, 2)


# speedup vs baseline: 10.3088x; 1.0088x over previous
import jax
import jax.numpy as jnp
from jax import lax
from jax.experimental import pallas as pl
from jax.experimental.pallas import tpu as pltpu
from jax.experimental.pallas import tpu_sc as plsc

B_TOTAL = 4096 * 50
EMBED = 128
NW = 32
CHUNK = 256
PER_W = B_TOTAL // NW
N_CHUNKS = PER_W // CHUNK
NBUF = 3


def _embed_kernel(ids_hbm, table_hbm, out_hbm, idx_v, rows_v, *sems):
    gsems, osems = sems[:NBUF], sems[NBUF:]
    wid = lax.axis_index("s") * 2 + lax.axis_index("c")
    base = wid * PER_W
    pltpu.sync_copy(ids_hbm.at[pl.ds(base, PER_W)], idx_v)

    def gather(j, b):
        return pltpu.make_async_copy(
            table_hbm.at[idx_v.at[pl.ds(j * CHUNK, CHUNK)]],
            rows_v.at[b],
            gsems[b],
        )

    def writeback(j, b):
        return pltpu.make_async_copy(
            rows_v.at[b],
            out_hbm.at[pl.ds(base + j * CHUNK, CHUNK)],
            osems[b],
        )

    for j in range(NBUF):
        gather(j, j).start()
    for j in range(N_CHUNKS):
        b = j % NBUF
        gather(j, b).wait()
        writeback(j, b).start()
        if j + NBUF < N_CHUNKS:
            writeback(j, b).wait()
            gather(j + NBUF, b).start()
    for j in range(N_CHUNKS - NBUF, N_CHUNKS):
        writeback(j, j % NBUF).wait()


@jax.jit
def _embed(ids_flat, table):
    mesh = plsc.VectorSubcoreMesh(core_axis_name="c", subcore_axis_name="s")
    return pl.kernel(
        _embed_kernel,
        out_type=jax.ShapeDtypeStruct((B_TOTAL, EMBED), jnp.float32),
        mesh=mesh,
        scratch_types=[
            pltpu.VMEM((PER_W,), jnp.int32),
            pltpu.VMEM((NBUF, CHUNK, EMBED), jnp.float32),
        ]
        + [pltpu.SemaphoreType.DMA] * (2 * NBUF),
    )(ids_flat, table)


def kernel(ids, W):
    batch, seq = ids.shape
    ids_t = ids.T.reshape(B_TOTAL).astype(jnp.int32)
    out = _embed(ids_t, W)
    return out.reshape(seq, batch, EMBED).transpose(1, 0, 2)
